# Initial kernel scaffold; baseline (speedup 1.0000x reference)
import jax
import jax.numpy as jnp
from jax.experimental import pallas as pl


def kernel(pos_rw, neg_rw, embedding):
    raise NotImplementedError("write your pallas kernel here")


# speedup vs baseline: 1.5433x; 1.5433x over previous
import functools

import jax
import jax.numpy as jnp
from jax import lax
from jax.experimental import pallas as pl
from jax.experimental.pallas import tpu as pltpu
from jax.experimental.pallas import tpu_sc as plsc

NUM_NODES = 100000
D = 128
DV = D // 16
W = 16384
C = 10
R = C - 1
EPS = 1e-15
NC = 2
NS = 16
NW = NC * NS
WPW = W // NW
CHUNK = 16
HIDX = CHUNK * C // 2
NCHUNK = WPW // CHUNK
CDOTS = CHUNK * R
DOTS = W * R
DPW = WPW * R


def _sc_body(pos_hbm, neg_hbm, emb_hbm, out_hbm,
             idx_a, idx_b, rows_v, part_v, dots_v, sem):
    cid = lax.axis_index("c")
    sid = lax.axis_index("s")
    wid = sid * NC + cid
    walk_base = wid * WPW
    lane = jax.lax.iota(jnp.int32, 16)

    for term, rw_hbm in enumerate((pos_hbm, neg_hbm)):
        def chunk_body(ch, carry):
            idx_off = (walk_base + ch * CHUNK) * C
            pltpu.sync_copy(rw_hbm.at[pl.ds(idx_off, HIDX)], idx_a)
            pltpu.sync_copy(rw_hbm.at[pl.ds(idx_off + HIDX, HIDX)], idx_b)
            cp1 = pltpu.async_copy(
                emb_hbm.at[idx_a], rows_v.at[pl.ds(0, HIDX)], sem)
            cp2 = pltpu.async_copy(
                emb_hbm.at[idx_b], rows_v.at[pl.ds(HIDX, HIDX)], sem)
            cp1.wait()
            cp2.wait()

            def walk_body(w, carry2):
                row0 = w * C
                s = [rows_v[row0, pl.ds(k * 16, 16)] for k in range(DV)]
                for j in range(R):
                    rrow = row0 + 1 + j
                    acc = s[0] * rows_v[rrow, pl.ds(0, 16)]
                    for k in range(1, DV):
                        acc = acc + s[k] * rows_v[rrow, pl.ds(k * 16, 16)]
                    part_v[pl.ds((w * R + j) * 16, 16)] = acc
                return carry2

            lax.fori_loop(0, CHUNK, walk_body, 0)

            for g in range(CDOTS // 16):
                rows_idx = (lane + (16 * g)) * 16
                tot = plsc.load_gather(part_v, [rows_idx])
                for c in range(1, 16):
                    tot = tot + plsc.load_gather(part_v, [rows_idx + c])
                dots_v[pl.ds(ch * CDOTS + 16 * g, 16)] = tot
            return carry

        lax.fori_loop(0, NCHUNK, chunk_body, 0)
        pltpu.sync_copy(
            dots_v, out_hbm.at[pl.ds(term * DOTS + walk_base * R, DPW)])


def _make_sc_dots():
    return pl.kernel(
        _sc_body,
        out_type=jax.ShapeDtypeStruct((2 * DOTS,), jnp.float32),
        mesh=plsc.VectorSubcoreMesh(
            core_axis_name="c", subcore_axis_name="s"),
        compiler_params=pltpu.CompilerParams(needs_layout_passes=False),
        scratch_types=[
            pltpu.VMEM((HIDX,), jnp.int32),
            pltpu.VMEM((HIDX,), jnp.int32),
            pltpu.VMEM((CHUNK * C, D), jnp.float32),
            pltpu.VMEM((CDOTS * 16,), jnp.float32),
            pltpu.VMEM((DPW,), jnp.float32),
            pltpu.SemaphoreType.DMA,
        ],
    )


_ROWS = 2 * DOTS // 128
_HALF = _ROWS // 2


def _tc_loss_body(dots_ref, out_ref):
    x = dots_ref[...]
    sig = 1.0 / (1.0 + jnp.exp(-x))
    pos = sig[:_HALF]
    neg = sig[_HALF:]
    pos_loss = -jnp.sum(jnp.log(pos + EPS))
    neg_loss = -jnp.sum(jnp.log(1.0 - neg + EPS))
    out_ref[0, 0] = (pos_loss + neg_loss) * (1.0 / DOTS)


_tc_loss = pl.pallas_call(
    _tc_loss_body,
    out_shape=jax.ShapeDtypeStruct((1, 1), jnp.float32),
    out_specs=pl.BlockSpec(memory_space=pltpu.SMEM),
)


def kernel(pos_rw, neg_rw, embedding):
    pos_flat = pos_rw.astype(jnp.int32).reshape(-1)
    neg_flat = neg_rw.astype(jnp.int32).reshape(-1)
    dots = _make_sc_dots()(pos_flat, neg_flat, embedding)
    loss = _tc_loss(dots.reshape(_ROWS, 128))
    return loss[0, 0]


# speedup vs baseline: 2.8570x; 1.8513x over previous
import functools

import jax
import jax.numpy as jnp
from jax import lax
from jax.experimental import pallas as pl
from jax.experimental.pallas import tpu as pltpu
from jax.experimental.pallas import tpu_sc as plsc

NUM_NODES = 100000
D = 128
DV = D // 16
W = 16384
C = 10
R = C - 1
EPS = 1e-15
NC = 2
NS = 16
NW = NC * NS
WPW = W // NW
CHUNK = 16
HIDX = CHUNK * C // 2
NCHUNK = WPW // CHUNK
CDOTS = CHUNK * R
DOTS = W * R
DPW = WPW * R


TCHUNK = 2 * NCHUNK


def _tree_sum(vs):
    while len(vs) > 1:
        vs = [a + b for a, b in zip(vs[::2], vs[1::2])]
    return vs[0]


def _sc_body(pos_hbm, neg_hbm, emb_hbm, out_hbm,
             idx_v, rows0, rows1, part_v, dots_v, sem0, sem1):
    cid = lax.axis_index("c")
    sid = lax.axis_index("s")
    wid = sid * NC + cid
    walk_base = wid * WPW
    lane = jax.lax.iota(jnp.int32, 16)

    pltpu.sync_copy(pos_hbm.at[pl.ds(walk_base * C, WPW * C)],
                    idx_v.at[pl.ds(0, WPW * C)])
    pltpu.sync_copy(neg_hbm.at[pl.ds(walk_base * C, WPW * C)],
                    idx_v.at[pl.ds(WPW * C, WPW * C)])

    rows = (rows0, rows1)
    sems = (sem0, sem1)

    def fetch(ch, buf):
        off = ch * (CHUNK * C)
        pltpu.async_copy(
            emb_hbm.at[idx_v.at[pl.ds(off, HIDX)]],
            rows[buf].at[pl.ds(0, HIDX)], sems[buf])
        pltpu.async_copy(
            emb_hbm.at[idx_v.at[pl.ds(off + HIDX, HIDX)]],
            rows[buf].at[pl.ds(HIDX, HIDX)], sems[buf])

    def wait_fetch(buf):
        pltpu.make_async_copy(
            emb_hbm.at[pl.ds(0, HIDX)],
            rows[buf].at[pl.ds(0, HIDX)], sems[buf]).wait()
        pltpu.make_async_copy(
            emb_hbm.at[pl.ds(0, HIDX)],
            rows[buf].at[pl.ds(HIDX, HIDX)], sems[buf]).wait()

    def compute(ch, buf):
        rows_v = rows[buf]

        def walk_body(w, carry2):
            row0 = w * C
            s = [rows_v[row0, pl.ds(k * 16, 16)] for k in range(DV)]
            for j in range(R):
                rrow = row0 + 1 + j
                acc = _tree_sum(
                    [s[k] * rows_v[rrow, pl.ds(k * 16, 16)]
                     for k in range(DV)])
                part_v[pl.ds((w * R + j) * 16, 16)] = acc
            return carry2

        lax.fori_loop(0, CHUNK, walk_body, 0)

        for g in range(CDOTS // 16):
            rows_idx = (lane + (16 * g)) * 16
            tot = _tree_sum(
                [plsc.load_gather(part_v, [rows_idx + c])
                 for c in range(16)])
            dots_v[pl.ds(ch * CDOTS + 16 * g, 16)] = tot

    fetch(0, 0)

    def chunk_pair(cp, carry):
        ch = 2 * cp
        fetch(ch + 1, 1)
        wait_fetch(0)
        compute(ch, 0)

        @pl.when(cp + 1 < TCHUNK // 2)
        def _():
            fetch(ch + 2, 0)
        wait_fetch(1)
        compute(ch + 1, 1)
        return carry

    lax.fori_loop(0, TCHUNK // 2, chunk_pair, 0)

    pltpu.sync_copy(dots_v.at[pl.ds(0, DPW)],
                    out_hbm.at[pl.ds(walk_base * R, DPW)])
    pltpu.sync_copy(dots_v.at[pl.ds(DPW, DPW)],
                    out_hbm.at[pl.ds(DOTS + walk_base * R, DPW)])


def _make_sc_dots():
    return pl.kernel(
        _sc_body,
        out_type=jax.ShapeDtypeStruct((2 * DOTS,), jnp.float32),
        mesh=plsc.VectorSubcoreMesh(
            core_axis_name="c", subcore_axis_name="s"),
        compiler_params=pltpu.CompilerParams(needs_layout_passes=False),
        scratch_types=[
            pltpu.VMEM((2 * WPW * C,), jnp.int32),
            pltpu.VMEM((CHUNK * C, D), jnp.float32),
            pltpu.VMEM((CHUNK * C, D), jnp.float32),
            pltpu.VMEM((CDOTS * 16,), jnp.float32),
            pltpu.VMEM((2 * DPW,), jnp.float32),
            pltpu.SemaphoreType.DMA,
            pltpu.SemaphoreType.DMA,
        ],
    )


_ROWS = 2 * DOTS // 128
_HALF = _ROWS // 2


def _tc_loss_body(dots_ref, out_ref):
    x = dots_ref[...]
    sig = 1.0 / (1.0 + jnp.exp(-x))
    pos = sig[:_HALF]
    neg = sig[_HALF:]
    pos_loss = -jnp.sum(jnp.log(pos + EPS))
    neg_loss = -jnp.sum(jnp.log(1.0 - neg + EPS))
    out_ref[0, 0] = (pos_loss + neg_loss) * (1.0 / DOTS)


_tc_loss = pl.pallas_call(
    _tc_loss_body,
    out_shape=jax.ShapeDtypeStruct((1, 1), jnp.float32),
    out_specs=pl.BlockSpec(memory_space=pltpu.SMEM),
)


def kernel(pos_rw, neg_rw, embedding):
    pos_flat = pos_rw.astype(jnp.int32).reshape(-1)
    neg_flat = neg_rw.astype(jnp.int32).reshape(-1)
    dots = _make_sc_dots()(pos_flat, neg_flat, embedding)
    loss = _tc_loss(dots.reshape(_ROWS, 128))
    return loss[0, 0]
